# baseline (speedup 1.0000x reference)
import functools

import jax
import jax.numpy as jnp
from jax import lax
from jax.experimental import pallas as pl
from jax.experimental.pallas import tpu as pltpu
from jax.experimental.pallas import tpu_sc as plsc

_N = 10000
_E = 320000
_D = 128

_NC = 2
_NS = 16
_NW = _NC * _NS
_LANES = 16

_CHUNK = 80
_NCH = -(-_E // (_NW * _CHUNK))
_EPT = _NCH * _CHUNK
_EPAD = _NW * _EPT
_ZR = 624
_TAIL = _N - _NS * _ZR

_BR = 1000


def _sc_aggregate(h2, src, dst, par3):
    mesh = plsc.VectorSubcoreMesh(core_axis_name="c", subcore_axis_name="s")

    @functools.partial(
        pl.kernel,
        out_type=jax.ShapeDtypeStruct((_NC, _N, _D), jnp.float32),
        mesh=mesh,
        scratch_types=[
            pltpu.VMEM((_CHUNK,), jnp.int32),
            pltpu.VMEM((_CHUNK,), jnp.int32),
            pltpu.VMEM((_CHUNK,), jnp.int32),
            pltpu.VMEM((_CHUNK,), jnp.int32),
            pltpu.VMEM((_NCH, _CHUNK), jnp.float32),
            pltpu.VMEM((_CHUNK, _D), jnp.float32),
            pltpu.VMEM((_CHUNK, _D), jnp.float32),
            pltpu.VMEM_SHARED((_N, _D), jnp.float32),
            pltpu.SemaphoreType.DMA,
            pltpu.SemaphoreType.DMA,
            pltpu.SemaphoreType.DMA,
            pltpu.SemaphoreType.DMA,
            pltpu.SemaphoreType.DMA,
            pltpu.SemaphoreType.DMA,
            pltpu.SemaphoreType.DMA,
            pltpu.SemaphoreType.DMA,
        ],
    )
    def agg(h2_hbm, src_hbm, dst_hbm, par3_hbm, out_hbm,
            srcv0, srcv1, dstv0, dstv1, parb, rows0, rows1, acc,
            isem0, isem1, dsem0, dsem1, gsem0, gsem1, ssem0, ssem1):
        c = lax.axis_index("c")
        s = lax.axis_index("s")
        wid = s * _NC + c
        base = wid * _EPT

        srcv = (srcv0, srcv1)
        dstv = (dstv0, dstv1)
        rows = (rows0, rows1)
        isem = (isem0, isem1)
        dsem = (dsem0, dsem1)
        gsem = (gsem0, gsem1)
        ssem = (ssem0, ssem1)

        pltpu.sync_copy(par3_hbm.at[wid], parb)

        zero = jnp.zeros((_LANES,), jnp.float32)

        @pl.loop(0, _CHUNK)
        def _(r):
            for j in range(_D // _LANES):
                rows0[r, pl.ds(j * _LANES, _LANES)] = zero

        row0 = s * _ZR
        for z in range(_ZR // _CHUNK):
            pltpu.sync_copy(rows0, acc.at[pl.ds(row0 + z * _CHUNK, _CHUNK)])
        rem = _ZR - (_ZR // _CHUNK) * _CHUNK
        if rem:
            pltpu.sync_copy(rows0.at[pl.ds(0, rem)],
                            acc.at[pl.ds(row0 + _ZR - rem, rem)])

        @pl.when(s == 0)
        def _():
            pltpu.sync_copy(rows0.at[pl.ds(0, _TAIL)],
                            acc.at[pl.ds(_NS * _ZR, _TAIL)])

        plsc.subcore_barrier()

        def issue_srci(i, b):
            pltpu.async_copy(
                src_hbm.at[pl.ds(base + i * _CHUNK, _CHUNK)], srcv[b], isem[b])

        def issue_dst(i, b):
            pltpu.async_copy(
                dst_hbm.at[pl.ds(base + i * _CHUNK, _CHUNK)], dstv[b], dsem[b])

        def wait_srci(b):
            pltpu.make_async_copy(
                src_hbm.at[pl.ds(base, _CHUNK)], srcv[b], isem[b]).wait()

        def wait_dst(b):
            pltpu.make_async_copy(
                src_hbm.at[pl.ds(base, _CHUNK)], dstv[b], dsem[b]).wait()

        def issue_gather(b):
            pltpu.async_copy(h2_hbm.at[srcv[b]], rows[b], gsem[b])

        def wait_gather(b):
            pltpu.make_async_copy(h2_hbm.at[srcv[b]], rows[b], gsem[b]).wait()

        def issue_scatter(i, b):
            pltpu.async_copy(rows[b], acc.at[dstv[b]], ssem[b], add=True)

        def wait_scatter(b):
            pltpu.make_async_copy(rows[b], acc.at[dstv[b]], ssem[b]).wait()

        def scale(i, b):
            return
            rb = rows[b]

            @pl.loop(0, _CHUNK // _LANES)
            def _(g):
                e0 = g * _LANES
                pg = parb[i, pl.ds(e0, _LANES)]
                for k in range(_LANES):
                    pvec = jnp.full((_LANES,), pg[k], jnp.float32)
                    for j in range(_D // _LANES):
                        sl = pl.ds(j * _LANES, _LANES)
                        rb[e0 + k, sl] = rb[e0 + k, sl] * pvec

        issue_srci(0, 0)
        issue_srci(1, 1)
        issue_dst(0, 0)
        issue_dst(1, 1)
        wait_srci(0)
        issue_gather(0)

        @pl.loop(0, _NCH - 1, step=2)
        def _(i):
            wait_gather(0)
            wait_srci(1)

            @pl.when(i > 0)
            def _():
                wait_scatter(1)
                issue_dst(i + 1, 1)

            issue_gather(1)
            issue_srci(i + 2, 0)
            scale(i, 0)
            wait_dst(0)
            issue_scatter(0, 0)

            wait_gather(1)
            wait_srci(0)
            wait_scatter(0)
            issue_dst(i + 2, 0)
            issue_gather(0)

            @pl.when(i < _NCH - 3)
            def _():
                issue_srci(i + 3, 1)

            scale(i + 1, 1)
            wait_dst(1)
            issue_scatter(0, 1)

        wait_gather(0)
        scale(_NCH - 1, 0)
        wait_dst(0)
        issue_scatter(0, 0)
        wait_scatter(0)
        wait_scatter(1)

        plsc.subcore_barrier()
        pltpu.sync_copy(acc.at[pl.ds(row0, _ZR)],
                        out_hbm.at[c, pl.ds(row0, _ZR)])

        @pl.when(s == 0)
        def _():
            pltpu.sync_copy(acc.at[pl.ds(_NS * _ZR, _TAIL)],
                            out_hbm.at[c, pl.ds(_NS * _ZR, _TAIL)])

    return agg(h2, src, dst, par3)


def _dense(parts, W, b, relu):
    P = parts.shape[0]

    def body(a_ref, w_ref, b_ref, o_ref):
        h = a_ref[0]
        for p in range(1, P):
            h = h + a_ref[p]
        if relu:
            h = jnp.maximum(h, 0.0)
        o_ref[...] = jnp.dot(h, w_ref[...],
                             preferred_element_type=jnp.float32) + b_ref[...]

    return pl.pallas_call(
        body,
        grid=(_N // _BR,),
        in_specs=[
            pl.BlockSpec((P, _BR, _D), lambda i: (0, i, 0)),
            pl.BlockSpec((_D, _D), lambda i: (0, 0)),
            pl.BlockSpec((1, _D), lambda i: (0, 0)),
        ],
        out_specs=pl.BlockSpec((_BR, _D), lambda i: (i, 0)),
        out_shape=jax.ShapeDtypeStruct((_N, _D), jnp.float32),
    )(parts, W, b.reshape(1, _D))


def _relu_softmax(parts):
    def body(a_ref, o_ref):
        h = jnp.maximum(a_ref[0] + a_ref[1], 0.0)
        m = jnp.max(h, axis=-1, keepdims=True)
        ex = jnp.exp(h - m)
        o_ref[...] = ex / jnp.sum(ex, axis=-1, keepdims=True)

    return pl.pallas_call(
        body,
        grid=(_N // _BR,),
        in_specs=[pl.BlockSpec((_NC, _BR, _D), lambda i: (0, i, 0))],
        out_specs=pl.BlockSpec((_BR, _D), lambda i: (i, 0)),
        out_shape=jax.ShapeDtypeStruct((_N, _D), jnp.float32),
    )(parts)


def kernel(X, graph, param, W1, b1, W2, b2, W3, b3):
    pad = _EPAD - _E
    src = jnp.concatenate([graph[0], jnp.zeros((pad,), jnp.int32)])
    dst = jnp.concatenate([graph[1], jnp.zeros((pad,), jnp.int32)])
    par3 = jnp.concatenate([param, jnp.zeros((pad,), jnp.float32)])
    par3 = par3.reshape(_NW, _NCH, _CHUNK)

    h2 = _dense(X[None], W1, b1, relu=False)
    a = _sc_aggregate(h2, src, dst, par3)
    h2 = _dense(a, W2, b2, relu=True)
    a = _sc_aggregate(h2, src, dst, par3)
    h2 = _dense(a, W3, b3, relu=True)
    a = _sc_aggregate(h2, src, dst, par3)
    return _relu_softmax(a)


# speedup vs baseline: 1.0933x; 1.0933x over previous
import functools

import jax
import jax.numpy as jnp
from jax import lax
from jax.experimental import pallas as pl
from jax.experimental.pallas import tpu as pltpu
from jax.experimental.pallas import tpu_sc as plsc

_N = 10000
_E = 320000
_D = 128

_NC = 2
_NS = 16
_NW = _NC * _NS
_LANES = 16

_CHUNK = 80
_NCH = -(-_E // (_NW * _CHUNK))
_EPT = _NCH * _CHUNK
_EPAD = _NW * _EPT
_ZR = 624
_TAIL = _N - _NS * _ZR

_BR = 1000


def _sc_aggregate(h2, src, dst, par3):
    mesh = plsc.VectorSubcoreMesh(core_axis_name="c", subcore_axis_name="s")

    @functools.partial(
        pl.kernel,
        out_type=jax.ShapeDtypeStruct((_NC, _N, _D), jnp.float32),
        mesh=mesh,
        scratch_types=[
            pltpu.VMEM((_CHUNK,), jnp.int32),
            pltpu.VMEM((_CHUNK,), jnp.int32),
            pltpu.VMEM((_CHUNK,), jnp.int32),
            pltpu.VMEM((_CHUNK,), jnp.int32),
            pltpu.VMEM((_NCH, _CHUNK), jnp.float32),
            pltpu.VMEM((_CHUNK, _D), jnp.float32),
            pltpu.VMEM((_CHUNK, _D), jnp.float32),
            pltpu.VMEM_SHARED((_N, _D), jnp.float32),
            pltpu.SemaphoreType.DMA,
            pltpu.SemaphoreType.DMA,
            pltpu.SemaphoreType.DMA,
            pltpu.SemaphoreType.DMA,
            pltpu.SemaphoreType.DMA,
            pltpu.SemaphoreType.DMA,
            pltpu.SemaphoreType.DMA,
            pltpu.SemaphoreType.DMA,
        ],
    )
    def agg(h2_hbm, src_hbm, dst_hbm, par3_hbm, out_hbm,
            srcv0, srcv1, dstv0, dstv1, parb, rows0, rows1, acc,
            isem0, isem1, dsem0, dsem1, gsem0, gsem1, ssem0, ssem1):
        c = lax.axis_index("c")
        s = lax.axis_index("s")
        wid = s * _NC + c
        base = wid * _EPT

        srcv = (srcv0, srcv1)
        dstv = (dstv0, dstv1)
        rows = (rows0, rows1)
        isem = (isem0, isem1)
        dsem = (dsem0, dsem1)
        gsem = (gsem0, gsem1)
        ssem = (ssem0, ssem1)

        pltpu.sync_copy(par3_hbm.at[wid], parb)

        zero = jnp.zeros((_LANES,), jnp.float32)

        @pl.loop(0, _CHUNK)
        def _(r):
            for j in range(_D // _LANES):
                rows0[r, pl.ds(j * _LANES, _LANES)] = zero

        row0 = s * _ZR
        for z in range(_ZR // _CHUNK):
            pltpu.sync_copy(rows0, acc.at[pl.ds(row0 + z * _CHUNK, _CHUNK)])
        rem = _ZR - (_ZR // _CHUNK) * _CHUNK
        if rem:
            pltpu.sync_copy(rows0.at[pl.ds(0, rem)],
                            acc.at[pl.ds(row0 + _ZR - rem, rem)])

        @pl.when(s == 0)
        def _():
            pltpu.sync_copy(rows0.at[pl.ds(0, _TAIL)],
                            acc.at[pl.ds(_NS * _ZR, _TAIL)])

        plsc.subcore_barrier()

        def issue_srci(i, b):
            pltpu.async_copy(
                src_hbm.at[pl.ds(base + i * _CHUNK, _CHUNK)], srcv[b], isem[b])

        def issue_dst(i, b):
            pltpu.async_copy(
                dst_hbm.at[pl.ds(base + i * _CHUNK, _CHUNK)], dstv[b], dsem[b])

        def wait_srci(b):
            pltpu.make_async_copy(
                src_hbm.at[pl.ds(base, _CHUNK)], srcv[b], isem[b]).wait()

        def wait_dst(b):
            pltpu.make_async_copy(
                src_hbm.at[pl.ds(base, _CHUNK)], dstv[b], dsem[b]).wait()

        def issue_gather(b):
            pltpu.async_copy(h2_hbm.at[srcv[b]], rows[b], gsem[b])

        def wait_gather(b):
            pltpu.make_async_copy(h2_hbm.at[srcv[b]], rows[b], gsem[b]).wait()

        def issue_scatter(i, b):
            pltpu.async_copy(rows[b], acc.at[dstv[b]], ssem[b], add=True)

        def wait_scatter(b):
            pltpu.make_async_copy(rows[b], acc.at[dstv[b]], ssem[b]).wait()

        def scale(i, b):
            rb = rows[b]

            @pl.loop(0, _CHUNK // _LANES)
            def _(g):
                e0 = g * _LANES
                pg = parb[i, pl.ds(e0, _LANES)]
                for k in range(_LANES):
                    pvec = jnp.full((_LANES,), pg[k], jnp.float32)
                    for j in range(_D // _LANES):
                        sl = pl.ds(j * _LANES, _LANES)
                        rb[e0 + k, sl] = rb[e0 + k, sl] * pvec

        issue_srci(0, 0)
        issue_srci(1, 1)
        issue_dst(0, 0)
        issue_dst(1, 1)
        wait_srci(0)
        issue_gather(0)

        @pl.loop(0, _NCH - 1, step=2)
        def _(i):
            wait_srci(1)

            @pl.when(i > 0)
            def _():
                wait_scatter(1)
                issue_dst(i + 1, 1)

            issue_gather(1)
            wait_gather(0)
            issue_srci(i + 2, 0)
            scale(i, 0)
            wait_dst(0)
            issue_scatter(0, 0)

            wait_srci(0)
            wait_scatter(0)
            issue_dst(i + 2, 0)
            issue_gather(0)
            wait_gather(1)

            @pl.when(i < _NCH - 3)
            def _():
                issue_srci(i + 3, 1)

            scale(i + 1, 1)
            wait_dst(1)
            issue_scatter(0, 1)

        wait_gather(0)
        scale(_NCH - 1, 0)
        wait_dst(0)
        issue_scatter(0, 0)
        wait_scatter(0)
        wait_scatter(1)

        plsc.subcore_barrier()
        pltpu.sync_copy(acc.at[pl.ds(row0, _ZR)],
                        out_hbm.at[c, pl.ds(row0, _ZR)])

        @pl.when(s == 0)
        def _():
            pltpu.sync_copy(acc.at[pl.ds(_NS * _ZR, _TAIL)],
                            out_hbm.at[c, pl.ds(_NS * _ZR, _TAIL)])

    return agg(h2, src, dst, par3)


def _dense(parts, W, b, relu):
    P = parts.shape[0]

    def body(a_ref, w_ref, b_ref, o_ref):
        h = a_ref[0]
        for p in range(1, P):
            h = h + a_ref[p]
        if relu:
            h = jnp.maximum(h, 0.0)
        o_ref[...] = jnp.dot(h, w_ref[...],
                             preferred_element_type=jnp.float32) + b_ref[...]

    return pl.pallas_call(
        body,
        grid=(_N // _BR,),
        in_specs=[
            pl.BlockSpec((P, _BR, _D), lambda i: (0, i, 0)),
            pl.BlockSpec((_D, _D), lambda i: (0, 0)),
            pl.BlockSpec((1, _D), lambda i: (0, 0)),
        ],
        out_specs=pl.BlockSpec((_BR, _D), lambda i: (i, 0)),
        out_shape=jax.ShapeDtypeStruct((_N, _D), jnp.float32),
    )(parts, W, b.reshape(1, _D))


def _relu_softmax(parts):
    def body(a_ref, o_ref):
        h = jnp.maximum(a_ref[0] + a_ref[1], 0.0)
        m = jnp.max(h, axis=-1, keepdims=True)
        ex = jnp.exp(h - m)
        o_ref[...] = ex / jnp.sum(ex, axis=-1, keepdims=True)

    return pl.pallas_call(
        body,
        grid=(_N // _BR,),
        in_specs=[pl.BlockSpec((_NC, _BR, _D), lambda i: (0, i, 0))],
        out_specs=pl.BlockSpec((_BR, _D), lambda i: (i, 0)),
        out_shape=jax.ShapeDtypeStruct((_N, _D), jnp.float32),
    )(parts)


def kernel(X, graph, param, W1, b1, W2, b2, W3, b3):
    pad = _EPAD - _E
    src = jnp.concatenate([graph[0], jnp.zeros((pad,), jnp.int32)])
    dst = jnp.concatenate([graph[1], jnp.zeros((pad,), jnp.int32)])
    par3 = jnp.concatenate([param, jnp.zeros((pad,), jnp.float32)])
    par3 = par3.reshape(_NW, _NCH, _CHUNK)

    h2 = _dense(X[None], W1, b1, relu=False)
    a = _sc_aggregate(h2, src, dst, par3)
    h2 = _dense(a, W2, b2, relu=True)
    a = _sc_aggregate(h2, src, dst, par3)
    h2 = _dense(a, W3, b3, relu=True)
    a = _sc_aggregate(h2, src, dst, par3)
    return _relu_softmax(a)


# speedup vs baseline: 1.1061x; 1.0117x over previous
import functools

import jax
import jax.numpy as jnp
from jax import lax
from jax.experimental import pallas as pl
from jax.experimental.pallas import tpu as pltpu
from jax.experimental.pallas import tpu_sc as plsc

_N = 10000
_E = 320000
_D = 128

_NC = 2
_NS = 16
_NW = _NC * _NS
_LANES = 16

_CHUNK = 80
_NCH = -(-_E // (_NW * _CHUNK))
_EPT = _NCH * _CHUNK
_EPAD = _NW * _EPT
_ZR = 624
_TAIL = _N - _NS * _ZR

_BR = 1000


def _sc_aggregate(h2, src, dst, par3):
    mesh = plsc.VectorSubcoreMesh(core_axis_name="c", subcore_axis_name="s")

    @functools.partial(
        pl.kernel,
        out_type=jax.ShapeDtypeStruct((_NC, _N, _D), jnp.float32),
        mesh=mesh,
        scratch_types=[
            pltpu.VMEM((_CHUNK,), jnp.int32),
            pltpu.VMEM((_CHUNK,), jnp.int32),
            pltpu.VMEM((_CHUNK,), jnp.int32),
            pltpu.VMEM((_CHUNK,), jnp.int32),
            pltpu.VMEM((_NCH, _CHUNK), jnp.float32),
            pltpu.VMEM((_CHUNK, _D), jnp.float32),
            pltpu.VMEM((_CHUNK, _D), jnp.float32),
            pltpu.VMEM_SHARED((_N, _D), jnp.float32),
            pltpu.SemaphoreType.DMA,
            pltpu.SemaphoreType.DMA,
            pltpu.SemaphoreType.DMA,
            pltpu.SemaphoreType.DMA,
            pltpu.SemaphoreType.DMA,
            pltpu.SemaphoreType.DMA,
            pltpu.SemaphoreType.DMA,
            pltpu.SemaphoreType.DMA,
        ],
    )
    def agg(h2_hbm, src_hbm, dst_hbm, par3_hbm, out_hbm,
            srcv0, srcv1, dstv0, dstv1, parb, rows0, rows1, acc,
            isem0, isem1, dsem0, dsem1, gsem0, gsem1, ssem0, ssem1):
        c = lax.axis_index("c")
        s = lax.axis_index("s")
        wid = s * _NC + c
        base = wid * _EPT

        srcv = (srcv0, srcv1)
        dstv = (dstv0, dstv1)
        rows = (rows0, rows1)
        isem = (isem0, isem1)
        dsem = (dsem0, dsem1)
        gsem = (gsem0, gsem1)
        ssem = (ssem0, ssem1)

        pltpu.async_copy(par3_hbm.at[wid], parb, gsem1)

        zero = jnp.zeros((_LANES,), jnp.float32)

        @pl.loop(0, _CHUNK)
        def _(r):
            for j in range(_D // _LANES):
                rows0[r, pl.ds(j * _LANES, _LANES)] = zero

        row0 = s * _ZR
        for z in range(_ZR // _CHUNK):
            pltpu.async_copy(rows0, acc.at[pl.ds(row0 + z * _CHUNK, _CHUNK)],
                             gsem0)
        rem = _ZR - (_ZR // _CHUNK) * _CHUNK
        if rem:
            pltpu.async_copy(rows0.at[pl.ds(0, rem)],
                             acc.at[pl.ds(row0 + _ZR - rem, rem)], gsem0)

        @pl.when(s == 0)
        def _():
            pltpu.sync_copy(rows0.at[pl.ds(0, _TAIL)],
                            acc.at[pl.ds(_NS * _ZR, _TAIL)])

        for z in range(_ZR // _CHUNK):
            pltpu.make_async_copy(
                rows0, acc.at[pl.ds(row0 + z * _CHUNK, _CHUNK)], gsem0).wait()
        if rem:
            pltpu.make_async_copy(
                rows0.at[pl.ds(0, rem)],
                acc.at[pl.ds(row0 + _ZR - rem, rem)], gsem0).wait()
        pltpu.make_async_copy(par3_hbm.at[wid], parb, gsem1).wait()

        plsc.subcore_barrier()

        def issue_srci(i, b):
            pltpu.async_copy(
                src_hbm.at[pl.ds(base + i * _CHUNK, _CHUNK)], srcv[b], isem[b])

        def issue_dst(i, b):
            pltpu.async_copy(
                dst_hbm.at[pl.ds(base + i * _CHUNK, _CHUNK)], dstv[b], dsem[b])

        def wait_srci(b):
            pltpu.make_async_copy(
                src_hbm.at[pl.ds(base, _CHUNK)], srcv[b], isem[b]).wait()

        def wait_dst(b):
            pltpu.make_async_copy(
                src_hbm.at[pl.ds(base, _CHUNK)], dstv[b], dsem[b]).wait()

        def issue_gather(b):
            pltpu.async_copy(h2_hbm.at[srcv[b]], rows[b], gsem[b])

        def wait_gather(b):
            pltpu.make_async_copy(h2_hbm.at[srcv[b]], rows[b], gsem[b]).wait()

        def issue_scatter(i, b):
            pltpu.async_copy(rows[b], acc.at[dstv[b]], ssem[b], add=True)

        def wait_scatter(b):
            pltpu.make_async_copy(rows[b], acc.at[dstv[b]], ssem[b]).wait()

        def scale(i, b):
            rb = rows[b]

            @pl.loop(0, _CHUNK // _LANES)
            def _(g):
                e0 = g * _LANES
                pg = parb[i, pl.ds(e0, _LANES)]
                for k in range(_LANES):
                    pvec = jnp.full((_LANES,), pg[k], jnp.float32)
                    for j in range(_D // _LANES):
                        sl = pl.ds(j * _LANES, _LANES)
                        rb[e0 + k, sl] = rb[e0 + k, sl] * pvec

        issue_srci(0, 0)
        issue_srci(1, 1)
        issue_dst(0, 0)
        issue_dst(1, 1)
        wait_srci(0)
        issue_gather(0)

        @pl.loop(0, _NCH - 1, step=2)
        def _(i):
            wait_srci(1)

            @pl.when(i > 0)
            def _():
                wait_scatter(1)
                issue_dst(i + 1, 1)

            issue_gather(1)
            wait_gather(0)
            issue_srci(i + 2, 0)
            scale(i, 0)
            wait_dst(0)
            issue_scatter(0, 0)

            wait_srci(0)
            wait_scatter(0)
            issue_dst(i + 2, 0)
            issue_gather(0)
            wait_gather(1)

            @pl.when(i < _NCH - 3)
            def _():
                issue_srci(i + 3, 1)

            scale(i + 1, 1)
            wait_dst(1)
            issue_scatter(0, 1)

        wait_gather(0)
        scale(_NCH - 1, 0)
        wait_dst(0)
        issue_scatter(0, 0)
        wait_scatter(0)
        wait_scatter(1)

        plsc.subcore_barrier()
        pltpu.sync_copy(acc.at[pl.ds(row0, _ZR)],
                        out_hbm.at[c, pl.ds(row0, _ZR)])

        @pl.when(s == 0)
        def _():
            pltpu.sync_copy(acc.at[pl.ds(_NS * _ZR, _TAIL)],
                            out_hbm.at[c, pl.ds(_NS * _ZR, _TAIL)])

    return agg(h2, src, dst, par3)


def _dense(parts, W, b, relu):
    P = parts.shape[0]

    def body(a_ref, w_ref, b_ref, o_ref):
        h = a_ref[0]
        for p in range(1, P):
            h = h + a_ref[p]
        if relu:
            h = jnp.maximum(h, 0.0)
        o_ref[...] = jnp.dot(h, w_ref[...],
                             preferred_element_type=jnp.float32) + b_ref[...]

    return pl.pallas_call(
        body,
        grid=(_N // _BR,),
        in_specs=[
            pl.BlockSpec((P, _BR, _D), lambda i: (0, i, 0)),
            pl.BlockSpec((_D, _D), lambda i: (0, 0)),
            pl.BlockSpec((1, _D), lambda i: (0, 0)),
        ],
        out_specs=pl.BlockSpec((_BR, _D), lambda i: (i, 0)),
        out_shape=jax.ShapeDtypeStruct((_N, _D), jnp.float32),
    )(parts, W, b.reshape(1, _D))


def _relu_softmax(parts):
    def body(a_ref, o_ref):
        h = jnp.maximum(a_ref[0] + a_ref[1], 0.0)
        m = jnp.max(h, axis=-1, keepdims=True)
        ex = jnp.exp(h - m)
        o_ref[...] = ex / jnp.sum(ex, axis=-1, keepdims=True)

    return pl.pallas_call(
        body,
        grid=(_N // _BR,),
        in_specs=[pl.BlockSpec((_NC, _BR, _D), lambda i: (0, i, 0))],
        out_specs=pl.BlockSpec((_BR, _D), lambda i: (i, 0)),
        out_shape=jax.ShapeDtypeStruct((_N, _D), jnp.float32),
    )(parts)


def kernel(X, graph, param, W1, b1, W2, b2, W3, b3):
    pad = _EPAD - _E
    src = jnp.concatenate([graph[0], jnp.zeros((pad,), jnp.int32)])
    dst = jnp.concatenate([graph[1], jnp.zeros((pad,), jnp.int32)])
    par3 = jnp.concatenate([param, jnp.zeros((pad,), jnp.float32)])
    par3 = par3.reshape(_NW, _NCH, _CHUNK)

    h2 = _dense(X[None], W1, b1, relu=False)
    a = _sc_aggregate(h2, src, dst, par3)
    h2 = _dense(a, W2, b2, relu=True)
    a = _sc_aggregate(h2, src, dst, par3)
    h2 = _dense(a, W3, b3, relu=True)
    a = _sc_aggregate(h2, src, dst, par3)
    return _relu_softmax(a)
